# baseline (speedup 1.0000x reference)
import functools

import jax
import jax.numpy as jnp
from jax import lax
from jax.experimental import pallas as pl
from jax.experimental.pallas import tpu as pltpu
from jax.experimental.pallas import tpu_sc as plsc

D_TAB = 28
D_SH = 4
D_OUT = D_TAB + D_SH
NC = 2
NS = 16
L = 16
NW = NC * NS


def _build(batch, n_chunk_rows):
    bpw = batch // NW
    nchunk = 4 * bpw
    mesh = plsc.VectorSubcoreMesh(core_axis_name="c", subcore_axis_name="s")

    @functools.partial(
        pl.kernel,
        out_type=jax.ShapeDtypeStruct((batch, D_OUT), jnp.float32),
        mesh=mesh,
        compiler_params=pltpu.CompilerParams(
            use_tc_tiling_on_sc=False, needs_layout_passes=False),
        scratch_types=[
            pltpu.VMEM((bpw,), jnp.int32),
            pltpu.VMEM((nchunk,), jnp.int32),
            pltpu.VMEM((nchunk + 2, 8), jnp.float32),
            pltpu.VMEM((bpw, D_OUT), jnp.float32),
            pltpu.VMEM((L,), jnp.float32),
            pltpu.SemaphoreType.DMA,
        ],
    )
    def emb_kernel(x_hbm, w8_hbm, sh_hbm, out_hbm, idx_v, cidx_v, chunks,
                   rows, shv, sem):
        wid = lax.axis_index("s") * NC + lax.axis_index("c")
        base = wid * bpw
        pltpu.sync_copy(x_hbm.at[pl.ds(base, bpw)], idx_v)
        pltpu.sync_copy(sh_hbm, shv)
        lane = lax.iota(jnp.int32, L)

        def cbody(g, c):
            p = L * g + lane
            xg = plsc.load_gather(idx_v, [p >> 2])
            cidx_v[pl.ds(L * g, L)] = ((7 * xg) >> 1) + (p & 3)
            return c

        lax.fori_loop(0, nchunk // L, cbody, 0)
        gather = pltpu.async_copy(
            w8_hbm.at[cidx_v], chunks.at[pl.ds(0, nchunk)], sem)
        sval = shv[...]
        gather.wait()

        def abody(r, c):
            xv = plsc.load_gather(idx_v, [jnp.full((L,), r, jnp.int32)])
            f1 = 32 * r + (xv & 1) * 4 + lane
            v1 = plsc.load_gather(chunks, [f1 >> 3, f1 & 7])
            f2 = f1 + L
            v2 = plsc.load_gather(chunks, [f2 >> 3, f2 & 7])
            v2 = jnp.where(lane >= 12, sval, v2)
            rows[r, pl.ds(0, L)] = v1
            rows[r, pl.ds(L, L)] = v2
            return c

        lax.fori_loop(0, bpw, abody, 0)
        pltpu.sync_copy(rows, out_hbm.at[pl.ds(base, bpw)])

    return emb_kernel


def kernel(x, W, shared):
    batch = x.shape[0]
    n_emb, d_tab = W.shape
    w8 = W.reshape(n_emb * d_tab // 8, 8)
    sh16 = jnp.tile(shared.reshape(D_SH), L // D_SH)
    out = _build(batch, w8.shape[0])(x.astype(jnp.int32), w8, sh16)
    return out[:, None, :]
